# baseline (speedup 1.0000x reference)
import functools

import jax
import jax.numpy as jnp
from jax import lax
from jax.experimental import pallas as pl
from jax.experimental.pallas import tpu as pltpu
from jax.experimental.pallas import tpu_sc as plsc

B, V, L, D = 2, 50, 50, 128
VOCAB = 5000
N = B * V * 3 * L
E = 480000
NROW = 15104
KCH = NROW // 128
NC, NS = 2, 16
DW = 128
HALF = NROW // 2
DUMMY = HALF
ACCR = HALF + 128
EPAD = 483328
EPT = EPAD // NS
EH = 2
EKCH = EPT // EH // 128
RPT = HALF // NS
ZPT = ACCR // NS


@functools.cache
def _make_sc_gather_deg():
    mesh = plsc.VectorSubcoreMesh(core_axis_name="c", subcore_axis_name="s",
                                  num_cores=NC, num_subcores=NS)
    return functools.partial(
        pl.kernel,
        out_type=(jax.ShapeDtypeStruct((NROW, D), jnp.float32),
                  jax.ShapeDtypeStruct((NROW, DW), jnp.float32)),
        mesh=mesh,
        scratch_types=[
            pltpu.VMEM((128,), jnp.int32),
            pltpu.VMEM((EKCH, 128), jnp.int32),
            pltpu.VMEM((128, D), jnp.float32),
            pltpu.VMEM((128, DW), jnp.float32),
            pltpu.VMEM_SHARED((ACCR, DW), jnp.float32),
            pltpu.SemaphoreType.DMA,
        ],
    )(_sc_gather_deg_body)


def _sc_gather_deg_body(embs_hbm, nidx_hbm, dst_hbm, ones_hbm, zeros128_hbm,
                        nodes_hbm, deg_hbm,
                        nidx_v, didx_v, rows_v, ones_v, dega, sem):
    c = lax.axis_index("c")
    s = lax.axis_index("s")
    wid = s * NC + c
    pltpu.sync_copy(zeros128_hbm, dega.at[pl.ds(s * ZPT, ZPT)])

    def emb_chunk(i, _):
        j = wid + i * (NC * NS)

        @pl.when(j < KCH)
        def _():
            pltpu.sync_copy(nidx_hbm.at[j], nidx_v)
            pltpu.async_copy(embs_hbm.at[nidx_v], rows_v, sem).wait()
            pltpu.sync_copy(rows_v, nodes_hbm.at[pl.ds(j * 128, 128)])
        return 0

    lax.fori_loop(0, (KCH + NC * NS - 1) // (NC * NS), emb_chunk, 0)

    pltpu.sync_copy(ones_hbm, ones_v)
    plsc.subcore_barrier()
    for h in range(EH):
        pltpu.sync_copy(dst_hbm.at[c, s, h], didx_v)

        def deg_chunk(j, _):
            pltpu.sync_copy(ones_v, dega.at[didx_v.at[j]], add=True)
            return 0

        lax.fori_loop(0, EKCH, deg_chunk, 0)
    plsc.subcore_barrier()
    pltpu.sync_copy(dega.at[pl.ds(s * RPT, RPT)],
                    deg_hbm.at[pl.ds(c * HALF + s * RPT, RPT)])


@functools.cache
def _make_sc_edge_agg():
    mesh = plsc.VectorSubcoreMesh(core_axis_name="c", subcore_axis_name="s",
                                  num_cores=NC, num_subcores=NS)
    return functools.partial(
        pl.kernel,
        out_type=jax.ShapeDtypeStruct((NROW, D), jnp.float32),
        mesh=mesh,
        scratch_types=[
            pltpu.VMEM((EKCH, 128), jnp.int32),
            pltpu.VMEM((EKCH, 128), jnp.int32),
            pltpu.VMEM((128, D), jnp.float32),
            pltpu.VMEM((128, D), jnp.float32),
            pltpu.VMEM_SHARED((ACCR, D), jnp.float32),
            pltpu.SemaphoreType.DMA,
            pltpu.SemaphoreType.DMA,
        ],
    )(_sc_edge_agg_body)


def _sc_edge_agg_body(y_hbm, src_hbm, dst_hbm,
                      out_hbm, sidx_v, didx_v, rows0_v, rows1_v,
                      acc, sem0, sem1):
    c = lax.axis_index("c")
    s = lax.axis_index("s")
    pltpu.sync_copy(y_hbm.at[pl.ds(c * HALF + s * RPT, RPT)],
                    acc.at[pl.ds(s * RPT, RPT)])
    plsc.subcore_barrier()

    for h in range(EH):
        pltpu.sync_copy(src_hbm.at[s, h], sidx_v)
        pltpu.sync_copy(dst_hbm.at[c, s, h], didx_v)
        pltpu.async_copy(y_hbm.at[sidx_v.at[0]], rows0_v, sem0)

        def edge_pair(i, _):
            j = 2 * i
            cp1 = pltpu.async_copy(y_hbm.at[sidx_v.at[j + 1]], rows1_v, sem1)
            pltpu.make_async_copy(y_hbm.at[sidx_v.at[j]], rows0_v,
                                  sem0).wait()
            pltpu.sync_copy(rows0_v, acc.at[didx_v.at[j]], add=True)

            @pl.when(j + 2 < EKCH)
            def _():
                pltpu.async_copy(y_hbm.at[sidx_v.at[j + 2]], rows0_v, sem0)

            cp1.wait()
            pltpu.sync_copy(rows1_v, acc.at[didx_v.at[j + 1]], add=True)
            return 0

        lax.fori_loop(0, EKCH // 2, edge_pair, 0)
    plsc.subcore_barrier()
    pltpu.sync_copy(acc.at[pl.ds(s * RPT, RPT)],
                    out_hbm.at[pl.ds(c * HALF + s * RPT, RPT)])


def _tc1_body(nodes_ref, deg_ref, w_ref, y_ref, dinv_ref):
    d = lax.rsqrt(jnp.maximum(deg_ref[...][:, :1] + 1.0, 1.0))
    y = jnp.dot(nodes_ref[...], w_ref[...], preferred_element_type=jnp.float32)
    y_ref[...] = y * d
    dinv_ref[...] = jnp.broadcast_to(d, (128, 8))


def _tc1(nodes, deg, w1):
    return pl.pallas_call(
        _tc1_body,
        grid=(KCH,),
        in_specs=[
            pl.BlockSpec((128, D), lambda i: (i, 0)),
            pl.BlockSpec((128, DW), lambda i: (i, 0)),
            pl.BlockSpec((D, D), lambda i: (0, 0)),
        ],
        out_specs=[
            pl.BlockSpec((128, D), lambda i: (i, 0)),
            pl.BlockSpec((128, 8), lambda i: (i, 0)),
        ],
        out_shape=[jax.ShapeDtypeStruct((NROW, D), jnp.float32),
                   jax.ShapeDtypeStruct((NROW, 8), jnp.float32)],
    )(nodes, deg, w1)


def _tc2_body(agg_ref, dinv_ref, b_ref, w_ref, y_ref):
    d = dinv_ref[...][:, :1]
    h = agg_ref[...] * d + b_ref[...]
    h = jnp.maximum(h, 0.0)
    y = jnp.dot(h, w_ref[...], preferred_element_type=jnp.float32)
    y_ref[...] = y * d


def _tc2(agg, dinv, b1, w2):
    return pl.pallas_call(
        _tc2_body,
        grid=(KCH,),
        in_specs=[
            pl.BlockSpec((128, D), lambda i: (i, 0)),
            pl.BlockSpec((128, 8), lambda i: (i, 0)),
            pl.BlockSpec((1, D), lambda i: (0, 0)),
            pl.BlockSpec((D, D), lambda i: (0, 0)),
        ],
        out_specs=pl.BlockSpec((128, D), lambda i: (i, 0)),
        out_shape=jax.ShapeDtypeStruct((NROW, D), jnp.float32),
    )(agg, dinv, b1, w2)


def _tc3_body(agg_ref, dinv_ref, b2_ref,
              wih_c_ref, whh_c_ref, bih_c_ref, bhh_c_ref,
              wih_p_ref, whh_p_ref, bih_p_ref, bhh_p_ref,
              wih_d_ref, whh_d_ref, bih_d_ref, bhh_d_ref,
              fcw_ref, fcb_ref, out_ref, gi_scr):
    b2 = b2_ref[...]
    G = 3 * B

    def visit(v, accs):
        out = list(accs)
        for b in range(B):
            base = (b * V + v) * 3 * L
            ch = agg_ref[pl.ds(base, 3 * L), :]
            ch = ch * dinv_ref[pl.ds(base, 3 * L), :1] + b2
            for f in range(3):
                out[b * 3 + f] = out[b * 3 + f] + ch[f * L:(f + 1) * L, :]
        return tuple(out)

    accs0 = tuple(jnp.zeros((L, D), jnp.float32) for _ in range(3 * B))
    accs = lax.fori_loop(0, V, visit, accs0)

    z0 = jnp.zeros((L, D), jnp.float32)
    parts = []
    for f in range(3):
        for b in range(B):
            blocks = [z0, z0, z0]
            blocks[f] = accs[b * 3 + f]
            parts.append(jnp.concatenate(blocks, axis=1).reshape(L, 1, 3 * D))
    zrow = jnp.zeros((L, 1, 3 * D), jnp.float32)
    parts += [zrow, zrow]
    xblk = jnp.concatenate(parts, axis=1)
    wih = jnp.concatenate([wih_c_ref[...], wih_p_ref[...], wih_d_ref[...]],
                          axis=0)
    whh = jnp.concatenate([whh_c_ref[...], whh_p_ref[...], whh_d_ref[...]],
                          axis=0)
    bih = jnp.concatenate([bih_c_ref[...]] * B + [bih_p_ref[...]] * B
                          + [bih_d_ref[...]] * B, axis=0)
    bhh = jnp.concatenate([bhh_c_ref[...]] * B + [bhh_p_ref[...]] * B
                          + [bhh_d_ref[...]] * B, axis=0)
    gi_scr[...] = jnp.dot(xblk.reshape(L * 8, 3 * D), wih,
                          preferred_element_type=jnp.float32)

    row_f = lax.broadcasted_iota(jnp.int32, (G, 3 * D), 0) // B
    lane_f = lax.broadcasted_iota(jnp.int32, (G, 3 * D), 1) // D
    blkmask = (row_f == lane_f).astype(jnp.float32)

    def step(t, h):
        h_blk = jnp.concatenate([h, h, h], axis=1) * blkmask
        gh = jnp.dot(h_blk, whh, preferred_element_type=jnp.float32) + bhh
        gi_t = gi_scr[pl.ds(pl.multiple_of(8 * t, 8), 8), :][:G] + bih
        r = jax.nn.sigmoid(gi_t[:, :D] + gh[:, :D])
        z = jax.nn.sigmoid(gi_t[:, D:2 * D] + gh[:, D:2 * D])
        nc = jnp.tanh(gi_t[:, 2 * D:] + r * gh[:, 2 * D:])
        return (1.0 - z) * nc + z * h

    h = lax.fori_loop(0, L, step, jnp.zeros((G, D), jnp.float32))

    patient = jnp.concatenate(
        [jnp.concatenate([h[f * B + b:f * B + b + 1] for f in range(3)],
                         axis=1) for b in range(B)], axis=0)
    patient = jnp.maximum(patient, 0.0)
    out_ref[...] = (jnp.dot(patient, fcw_ref[...],
                            preferred_element_type=jnp.float32) + fcb_ref[...])


def _tc3(agg2, dinv, b2, gru_ws, fcw_p, fcb_p):
    full = lambda shape: pl.BlockSpec(shape, lambda: tuple(0 for _ in shape))
    gru_specs, gru_args = [], []
    for (wih, whh, bih, bhh) in gru_ws:
        gru_specs += [full((D, 3 * D)), full((D, 3 * D)),
                      full((1, 3 * D)), full((1, 3 * D))]
        gru_args += [wih, whh, bih, bhh]
    return pl.pallas_call(
        _tc3_body,
        in_specs=[full((NROW, D)), full((NROW, 8)), full((1, D))]
        + gru_specs + [full((3 * D, D)), full((1, D))],
        out_specs=full((B, D)),
        out_shape=jax.ShapeDtypeStruct((B, D), jnp.float32),
        scratch_shapes=[pltpu.VMEM((8 * L, 3 * D), jnp.float32)],
    )(agg2, dinv, b2, *gru_args, fcw_p, fcb_p)


def kernel(x_conditions, x_procedures, x_drugs, edge_index,
           emb_cond, emb_proc, emb_drug,
           gcn1_w, gcn1_b, gcn2_w, gcn2_b, fc_w, fc_b,
           gru_c_wih, gru_c_whh, gru_c_bih, gru_c_bhh,
           gru_p_wih, gru_p_whh, gru_p_bih, gru_p_bhh,
           gru_d_wih, gru_d_whh, gru_d_bih, gru_d_bhh):
    f32 = jnp.float32
    embs = jnp.concatenate([emb_cond, emb_proc, emb_drug], axis=0)
    nidx = jnp.concatenate(
        [x_conditions, x_procedures + VOCAB, x_drugs + 2 * VOCAB],
        axis=2).reshape(-1).astype(jnp.int32)
    nidx = jnp.concatenate([nidx, jnp.zeros((NROW - N,), jnp.int32)])
    nidx = nidx.reshape(KCH, 128)

    src = edge_index[0].astype(jnp.int32)
    dst = edge_index[1].astype(jnp.int32)
    src = jnp.concatenate([src, jnp.zeros((EPAD - E,), jnp.int32)])
    dst = jnp.concatenate([dst, jnp.full((EPAD - E,), N, jnp.int32)])
    perm = jnp.argsort((dst >= HALF).astype(jnp.int32), stable=True)
    src = src[perm]
    dst = dst[perm]
    in0 = dst < HALF
    dst0 = jnp.where(in0, dst, DUMMY)
    dst1 = jnp.where(in0, DUMMY, dst - HALF)
    src_r = src.reshape(NS, EH, EKCH, 128)
    dst_r = jnp.stack([dst0, dst1]).reshape(NC, NS, EH, EKCH, 128)

    ones128 = jnp.ones((128, DW), f32)
    zeros128 = jnp.zeros((ZPT, DW), f32)

    sc_gather_deg = _make_sc_gather_deg()
    sc_edge_agg = _make_sc_edge_agg()
    nodes, deg = sc_gather_deg(embs, nidx, dst_r, ones128, zeros128)
    y1, dinv = _tc1(nodes, deg, gcn1_w)
    agg1 = sc_edge_agg(y1, src_r, dst_r)
    y2 = _tc2(agg1, dinv, gcn1_b.reshape(1, D), gcn2_w)
    agg2 = sc_edge_agg(y2, src_r, dst_r)

    gru_ws = [(gru_c_wih, gru_c_whh, gru_c_bih.reshape(1, -1), gru_c_bhh.reshape(1, -1)),
              (gru_p_wih, gru_p_whh, gru_p_bih.reshape(1, -1), gru_p_bhh.reshape(1, -1)),
              (gru_d_wih, gru_d_whh, gru_d_bih.reshape(1, -1), gru_d_bhh.reshape(1, -1))]
    fcw_p = jnp.zeros((3 * D, D), f32).at[:, :10].set(fc_w)
    fcb_p = jnp.zeros((1, D), f32).at[0, :10].set(fc_b)
    out2 = _tc3(agg2, dinv, gcn2_b.reshape(1, D), gru_ws, fcw_p, fcb_p)
    return out2[:, :10]


# speedup vs baseline: 1.2787x; 1.2787x over previous
import functools

import jax
import jax.numpy as jnp
from jax import lax
from jax.experimental import pallas as pl
from jax.experimental.pallas import tpu as pltpu
from jax.experimental.pallas import tpu_sc as plsc

B, V, L, D = 2, 50, 50, 128
VOCAB = 5000
N = B * V * 3 * L
E = 480000
NROW = 15104
KCH = NROW // 128
NC, NS = 2, 16
DW = 128
HALF = NROW // 2
DUMMY = HALF
ACCR = HALF + 128
EPAD = 483328
EPT = EPAD // NS
EH = 2
EKCH = EPT // EH // 128
RPT = HALF // NS
ZPT = ACCR // NS


@functools.cache
def _make_sc_gather_deg():
    mesh = plsc.VectorSubcoreMesh(core_axis_name="c", subcore_axis_name="s",
                                  num_cores=NC, num_subcores=NS)
    return functools.partial(
        pl.kernel,
        out_type=(jax.ShapeDtypeStruct((NROW, D), jnp.float32),
                  jax.ShapeDtypeStruct((NROW, DW), jnp.float32)),
        mesh=mesh,
        scratch_types=[
            pltpu.VMEM((128,), jnp.int32),
            pltpu.VMEM((EKCH, 128), jnp.int32),
            pltpu.VMEM((128, D), jnp.float32),
            pltpu.VMEM((128, DW), jnp.float32),
            pltpu.VMEM_SHARED((ACCR, DW), jnp.float32),
            pltpu.SemaphoreType.DMA,
        ],
    )(_sc_gather_deg_body)


def _sc_gather_deg_body(embs_hbm, nidx_hbm, dst_hbm, ones_hbm, zeros128_hbm,
                        nodes_hbm, deg_hbm,
                        nidx_v, didx_v, rows_v, ones_v, dega, sem):
    c = lax.axis_index("c")
    s = lax.axis_index("s")
    wid = s * NC + c
    pltpu.sync_copy(zeros128_hbm, dega.at[pl.ds(s * ZPT, ZPT)])

    def emb_chunk(i, _):
        j = wid + i * (NC * NS)

        @pl.when(j < KCH)
        def _():
            pltpu.sync_copy(nidx_hbm.at[j], nidx_v)
            pltpu.async_copy(embs_hbm.at[nidx_v], rows_v, sem).wait()
            pltpu.sync_copy(rows_v, nodes_hbm.at[pl.ds(j * 128, 128)])
        return 0

    lax.fori_loop(0, (KCH + NC * NS - 1) // (NC * NS), emb_chunk, 0)

    pltpu.sync_copy(ones_hbm, ones_v)
    plsc.subcore_barrier()
    for h in range(EH):
        pltpu.sync_copy(dst_hbm.at[c, s, h], didx_v)

        def deg_chunk(j, _):
            pltpu.sync_copy(ones_v, dega.at[didx_v.at[j]], add=True)
            return 0

        lax.fori_loop(0, EKCH, deg_chunk, 0)
    plsc.subcore_barrier()
    pltpu.sync_copy(dega.at[pl.ds(s * RPT, RPT)],
                    deg_hbm.at[pl.ds(c * HALF + s * RPT, RPT)])


@functools.cache
def _make_sc_edge_agg():
    mesh = plsc.VectorSubcoreMesh(core_axis_name="c", subcore_axis_name="s",
                                  num_cores=NC, num_subcores=NS)
    return functools.partial(
        pl.kernel,
        out_type=jax.ShapeDtypeStruct((NROW, D), jnp.float32),
        mesh=mesh,
        scratch_types=[
            pltpu.VMEM((EKCH, 128), jnp.int32),
            pltpu.VMEM((EKCH, 128), jnp.int32),
            pltpu.VMEM((128, D), jnp.float32),
            pltpu.VMEM((128, D), jnp.float32),
            pltpu.VMEM_SHARED((ACCR, D), jnp.float32),
            pltpu.SemaphoreType.DMA,
            pltpu.SemaphoreType.DMA,
        ],
    )(_sc_edge_agg_body)


def _sc_edge_agg_body(y_hbm, src_hbm, dst_hbm,
                      out_hbm, sidx_v, didx_v, rows0_v, rows1_v,
                      acc, sem0, sem1):
    c = lax.axis_index("c")
    s = lax.axis_index("s")
    pltpu.sync_copy(y_hbm.at[pl.ds(c * HALF + s * RPT, RPT)],
                    acc.at[pl.ds(s * RPT, RPT)])
    plsc.subcore_barrier()

    for h in range(EH):
        pltpu.sync_copy(src_hbm.at[s, h], sidx_v)
        pltpu.sync_copy(dst_hbm.at[c, s, h], didx_v)
        pltpu.async_copy(y_hbm.at[sidx_v.at[0]], rows0_v, sem0)

        def edge_pair(i, _):
            j = 2 * i
            cp1 = pltpu.async_copy(y_hbm.at[sidx_v.at[j + 1]], rows1_v, sem1)
            pltpu.make_async_copy(y_hbm.at[sidx_v.at[j]], rows0_v,
                                  sem0).wait()
            pltpu.sync_copy(rows0_v, acc.at[didx_v.at[j]], add=True)

            @pl.when(j + 2 < EKCH)
            def _():
                pltpu.async_copy(y_hbm.at[sidx_v.at[j + 2]], rows0_v, sem0)

            cp1.wait()
            pltpu.sync_copy(rows1_v, acc.at[didx_v.at[j + 1]], add=True)
            return 0

        lax.fori_loop(0, EKCH // 2, edge_pair, 0)
    plsc.subcore_barrier()
    pltpu.sync_copy(acc.at[pl.ds(s * RPT, RPT)],
                    out_hbm.at[pl.ds(c * HALF + s * RPT, RPT)])


def _tc1_body(nodes_ref, deg_ref, w_ref, y_ref, dinv_ref):
    d = lax.rsqrt(jnp.maximum(deg_ref[...][:, :1] + 1.0, 1.0))
    y = jnp.dot(nodes_ref[...], w_ref[...], preferred_element_type=jnp.float32)
    y_ref[...] = y * d
    dinv_ref[...] = jnp.broadcast_to(d, (128, 8))


def _tc1(nodes, deg, w1):
    return pl.pallas_call(
        _tc1_body,
        grid=(KCH,),
        in_specs=[
            pl.BlockSpec((128, D), lambda i: (i, 0)),
            pl.BlockSpec((128, DW), lambda i: (i, 0)),
            pl.BlockSpec((D, D), lambda i: (0, 0)),
        ],
        out_specs=[
            pl.BlockSpec((128, D), lambda i: (i, 0)),
            pl.BlockSpec((128, 8), lambda i: (i, 0)),
        ],
        out_shape=[jax.ShapeDtypeStruct((NROW, D), jnp.float32),
                   jax.ShapeDtypeStruct((NROW, 8), jnp.float32)],
    )(nodes, deg, w1)


def _tc2_body(agg_ref, dinv_ref, b_ref, w_ref, y_ref):
    d = dinv_ref[...][:, :1]
    h = agg_ref[...] * d + b_ref[...]
    h = jnp.maximum(h, 0.0)
    y = jnp.dot(h, w_ref[...], preferred_element_type=jnp.float32)
    y_ref[...] = y * d


def _tc2(agg, dinv, b1, w2):
    return pl.pallas_call(
        _tc2_body,
        grid=(KCH,),
        in_specs=[
            pl.BlockSpec((128, D), lambda i: (i, 0)),
            pl.BlockSpec((128, 8), lambda i: (i, 0)),
            pl.BlockSpec((1, D), lambda i: (0, 0)),
            pl.BlockSpec((D, D), lambda i: (0, 0)),
        ],
        out_specs=pl.BlockSpec((128, D), lambda i: (i, 0)),
        out_shape=jax.ShapeDtypeStruct((NROW, D), jnp.float32),
    )(agg, dinv, b1, w2)


def _tc3_body(agg_ref, dinv_ref, b2_ref,
              wih_c_ref, whh_c_ref, bih_c_ref, bhh_c_ref,
              wih_p_ref, whh_p_ref, bih_p_ref, bhh_p_ref,
              wih_d_ref, whh_d_ref, bih_d_ref, bhh_d_ref,
              fcw_ref, fcb_ref, out_ref, gi_scr):
    b2 = b2_ref[...]
    G = 3 * B

    def visit(v, accs):
        out = list(accs)
        for b in range(B):
            base = (b * V + v) * 3 * L
            ch = agg_ref[pl.ds(base, 3 * L), :]
            ch = ch * dinv_ref[pl.ds(base, 3 * L), :1] + b2
            for f in range(3):
                out[b * 3 + f] = out[b * 3 + f] + ch[f * L:(f + 1) * L, :]
        return tuple(out)

    accs0 = tuple(jnp.zeros((L, D), jnp.float32) for _ in range(3 * B))
    accs = lax.fori_loop(0, V, visit, accs0)

    z0 = jnp.zeros((L, D), jnp.float32)
    parts = []
    for f in range(3):
        for b in range(B):
            blocks = [z0, z0, z0]
            blocks[f] = accs[b * 3 + f]
            parts.append(jnp.concatenate(blocks, axis=1).reshape(L, 1, 3 * D))
    zrow = jnp.zeros((L, 1, 3 * D), jnp.float32)
    parts += [zrow, zrow]
    xblk = jnp.concatenate(parts, axis=1)
    wih = jnp.concatenate([wih_c_ref[...], wih_p_ref[...], wih_d_ref[...]],
                          axis=0)
    whh = jnp.concatenate([whh_c_ref[...], whh_p_ref[...], whh_d_ref[...]],
                          axis=0)
    bih = jnp.concatenate([bih_c_ref[...]] * B + [bih_p_ref[...]] * B
                          + [bih_d_ref[...]] * B, axis=0)
    bhh = jnp.concatenate([bhh_c_ref[...]] * B + [bhh_p_ref[...]] * B
                          + [bhh_d_ref[...]] * B, axis=0)
    gi_scr[...] = jnp.dot(xblk.reshape(L * 8, 3 * D), wih,
                          preferred_element_type=jnp.float32)

    row_f = lax.broadcasted_iota(jnp.int32, (G, 3 * D), 0) // B
    lane_f = lax.broadcasted_iota(jnp.int32, (G, 3 * D), 1) // D
    blkmask = (row_f == lane_f).astype(jnp.float32)

    def step(t, h):
        h_blk = jnp.concatenate([h, h, h], axis=1) * blkmask
        gh = jnp.dot(h_blk, whh, preferred_element_type=jnp.float32) + bhh
        gi_t = gi_scr[pl.ds(pl.multiple_of(8 * t, 8), 8), :][:G] + bih
        r = jax.nn.sigmoid(gi_t[:, :D] + gh[:, :D])
        z = jax.nn.sigmoid(gi_t[:, D:2 * D] + gh[:, D:2 * D])
        nc = jnp.tanh(gi_t[:, 2 * D:] + r * gh[:, 2 * D:])
        return (1.0 - z) * nc + z * h

    h = lax.fori_loop(0, L, step, jnp.zeros((G, D), jnp.float32))

    patient = jnp.concatenate(
        [jnp.concatenate([h[f * B + b:f * B + b + 1] for f in range(3)],
                         axis=1) for b in range(B)], axis=0)
    patient = jnp.maximum(patient, 0.0)
    out_ref[...] = (jnp.dot(patient, fcw_ref[...],
                            preferred_element_type=jnp.float32) + fcb_ref[...])


def _tc3(agg2, dinv, b2, gru_ws, fcw_p, fcb_p):
    full = lambda shape: pl.BlockSpec(shape, lambda: tuple(0 for _ in shape))
    gru_specs, gru_args = [], []
    for (wih, whh, bih, bhh) in gru_ws:
        gru_specs += [full((D, 3 * D)), full((D, 3 * D)),
                      full((1, 3 * D)), full((1, 3 * D))]
        gru_args += [wih, whh, bih, bhh]
    return pl.pallas_call(
        _tc3_body,
        in_specs=[full((NROW, D)), full((NROW, 8)), full((1, D))]
        + gru_specs + [full((3 * D, D)), full((1, D))],
        out_specs=full((B, D)),
        out_shape=jax.ShapeDtypeStruct((B, D), jnp.float32),
        scratch_shapes=[pltpu.VMEM((8 * L, 3 * D), jnp.float32)],
    )(agg2, dinv, b2, *gru_args, fcw_p, fcb_p)


def kernel(x_conditions, x_procedures, x_drugs, edge_index,
           emb_cond, emb_proc, emb_drug,
           gcn1_w, gcn1_b, gcn2_w, gcn2_b, fc_w, fc_b,
           gru_c_wih, gru_c_whh, gru_c_bih, gru_c_bhh,
           gru_p_wih, gru_p_whh, gru_p_bih, gru_p_bhh,
           gru_d_wih, gru_d_whh, gru_d_bih, gru_d_bhh):
    f32 = jnp.float32
    embs = jnp.concatenate([emb_cond, emb_proc, emb_drug], axis=0)
    nidx = jnp.concatenate(
        [x_conditions, x_procedures + VOCAB, x_drugs + 2 * VOCAB],
        axis=2).reshape(-1).astype(jnp.int32)
    nidx = jnp.concatenate([nidx, jnp.zeros((NROW - N,), jnp.int32)])
    nidx = nidx.reshape(KCH, 128)

    src = edge_index[0].astype(jnp.int32)
    dst = edge_index[1].astype(jnp.int32)
    src = jnp.concatenate([src, jnp.zeros((EPAD - E,), jnp.int32)])
    dst = jnp.concatenate([dst, jnp.full((EPAD - E,), N, jnp.int32)])
    in0 = dst < HALF
    dst0 = jnp.where(in0, dst, DUMMY)
    dst1 = jnp.where(in0, DUMMY, dst - HALF)
    src_r = src.reshape(NS, EH, EKCH, 128)
    dst_r = jnp.stack([dst0, dst1]).reshape(NC, NS, EH, EKCH, 128)

    ones128 = jnp.ones((128, DW), f32)
    zeros128 = jnp.zeros((ZPT, DW), f32)

    sc_gather_deg = _make_sc_gather_deg()
    sc_edge_agg = _make_sc_edge_agg()
    nodes, deg = sc_gather_deg(embs, nidx, dst_r, ones128, zeros128)
    y1, dinv = _tc1(nodes, deg, gcn1_w)
    agg1 = sc_edge_agg(y1, src_r, dst_r)
    y2 = _tc2(agg1, dinv, gcn1_b.reshape(1, D), gcn2_w)
    agg2 = sc_edge_agg(y2, src_r, dst_r)

    gru_ws = [(gru_c_wih, gru_c_whh, gru_c_bih.reshape(1, -1), gru_c_bhh.reshape(1, -1)),
              (gru_p_wih, gru_p_whh, gru_p_bih.reshape(1, -1), gru_p_bhh.reshape(1, -1)),
              (gru_d_wih, gru_d_whh, gru_d_bih.reshape(1, -1), gru_d_bhh.reshape(1, -1))]
    fcw_p = jnp.zeros((3 * D, D), f32).at[:, :10].set(fc_w)
    fcb_p = jnp.zeros((1, D), f32).at[0, :10].set(fc_b)
    out2 = _tc3(agg2, dinv, gcn2_b.reshape(1, D), gru_ws, fcw_p, fcb_p)
    return out2[:, :10]
